# baseline (speedup 1.0000x reference)
import functools

import jax
import jax.numpy as jnp
from jax import lax
from jax.experimental import pallas as pl
from jax.experimental.pallas import tpu as pltpu
from jax.experimental.pallas import tpu_sc as plsc

VOCAB = 23236
EMBED_DIM = 1728
BATCH = 32
SEQ = 512

_NC = 2
_NS = 16
_NW = _NC * _NS

_B = BATCH * SEQ
_BPW = _B // _NW
_S = 128
_NSCH = _BPW // _S
_NFULL = EMBED_DIM // 128
_TAIL0 = _NFULL * 128


def _transpose_block(g, t, ncols):
    lane = jax.lax.iota(jnp.int32, 16)

    def col(c, _):
        cvec = jnp.full((16,), c, jnp.int32)
        for r0 in range(0, _S, 16):
            t[c, pl.ds(r0, 16)] = plsc.load_gather(g, [r0 + lane, cvec])
        return _
    lax.fori_loop(0, ncols, col, 0)


def _gather_body(table_hbm, tail_hbm, ids_hbm, out_hbm, idx_v,
                 g0, g1, t0, t1, gsem0, gsem1, wsem0, wsem1):
    wid = lax.axis_index("s") * _NC + lax.axis_index("c")
    base = wid * _BPW

    pltpu.sync_copy(ids_hbm.at[pl.ds(base, _BPW)], idx_v)

    gbuf = (g0, g1)
    tbuf = (t0, t1)
    gsem = (gsem0, gsem1)
    wsem = (wsem0, wsem1)

    def src(s, j):
        idx = idx_v.at[pl.ds(pl.multiple_of(s * _S, 8), _S)]
        if j < _NFULL:
            return table_hbm.at[idx, pl.ds(j * 128, 128)]
        return tail_hbm.at[idx]

    def start_gather(s, j, b):
        pltpu.async_copy(src(s, j), gbuf[b], gsem[b])

    def wait_gather(s, j, b):
        pltpu.make_async_copy(src(s, j), gbuf[b], gsem[b]).wait()

    def out_slab(s, j, nrows):
        return out_hbm.at[wid, pl.ds(j * 128, nrows),
                          pl.ds(pl.multiple_of(s * _S, 128), _S)]

    def start_write(s, j, b):
        if j < _NFULL:
            pltpu.async_copy(tbuf[b], out_slab(s, j, 128), wsem[b])
        else:
            pltpu.async_copy(tbuf[b].at[pl.ds(0, 64)],
                             out_slab(s, j, 64), wsem[b])

    def wait_write(s, j, b):
        if j < _NFULL:
            pltpu.make_async_copy(tbuf[b], out_slab(s, j, 128),
                                  wsem[b]).wait()
        else:
            pltpu.make_async_copy(tbuf[b].at[pl.ds(0, 64)],
                                  out_slab(s, j, 64), wsem[b]).wait()

    start_gather(0, 0, 0)
    start_gather(0, 1, 1)

    def s_body(s, carry):
        for j in range(_NFULL + 1):
            b = j % 2
            wait_gather(s, j, b)
            if j >= 2:
                wait_write(s, j - 2, b)
            else:
                @pl.when(s > 0)
                def _prev_write(j=j, b=b):
                    wait_write(s - 1, j + _NFULL - 1, b)
            _transpose_block(gbuf[b], tbuf[b],
                             128 if j < _NFULL else 64)
            start_write(s, j, b)
            if j < _NFULL - 1:
                start_gather(s, j + 2, b)
            else:
                @pl.when(s + 1 < _NSCH)
                def _next_gather(j=j, b=b):
                    start_gather(s + 1, j - _NFULL + 1, b)
        return carry

    lax.fori_loop(0, _NSCH, s_body, 0)
    wait_write(_NSCH - 1, _NFULL - 1, (_NFULL - 1) % 2)
    wait_write(_NSCH - 1, _NFULL, _NFULL % 2)


@jax.jit
def _embed(ids_flat, font_table, tail_table):
    mesh = plsc.VectorSubcoreMesh(core_axis_name="c", subcore_axis_name="s")
    run = pl.kernel(
        _gather_body,
        out_type=jax.ShapeDtypeStruct((BATCH, EMBED_DIM, SEQ), jnp.float32),
        mesh=mesh,
        scratch_types=[
            pltpu.VMEM((_BPW,), jnp.int32),
            pltpu.VMEM((_S, 128), jnp.float32),
            pltpu.VMEM((_S, 128), jnp.float32),
            pltpu.VMEM((128, _S), jnp.float32),
            pltpu.VMEM((128, _S), jnp.float32),
            pltpu.SemaphoreType.DMA,
            pltpu.SemaphoreType.DMA,
            pltpu.SemaphoreType.DMA,
            pltpu.SemaphoreType.DMA,
        ],
        compiler_params=pltpu.CompilerParams(needs_layout_passes=False),
    )
    return run(font_table, tail_table, ids_flat)


def kernel(input_ids, font_table):
    ids_flat = input_ids.reshape(-1).astype(jnp.int32)
    tail_table = jnp.pad(font_table[:, _TAIL0:], ((0, 0), (0, 64)))
    out_t = _embed(ids_flat, font_table, tail_table)
    return out_t.transpose(0, 2, 1)


# speedup vs baseline: 3.3329x; 3.3329x over previous
import functools

import jax
import jax.numpy as jnp
from jax import lax
from jax.experimental import pallas as pl
from jax.experimental.pallas import tpu as pltpu
from jax.experimental.pallas import tpu_sc as plsc
from jax.experimental.layout import Format, Layout, with_layout_constraint

VOCAB = 23236
EMBED_DIM = 1728
BATCH = 32
SEQ = 512

_NC = 2
_NS = 16
_NW = _NC * _NS

_B = BATCH * SEQ
_BPW = _B // _NW
_K = 32
_NCH = _BPW // _K
_NFULL = EMBED_DIM // 128
_TAIL0 = _NFULL * 128


def _gather_body(table_hbm, tail_hbm, ids_hbm, out_hbm, idx_v,
                 rows0, rows1, tail0, tail1, gsem0, gsem1):
    wid = lax.axis_index("s") * _NC + lax.axis_index("c")
    base = wid * _BPW

    pltpu.sync_copy(ids_hbm.at[pl.ds(base, _BPW)], idx_v)

    def start_gathers(c, rows, tail, sem):
        idx = idx_v.at[pl.ds(c * _K, _K)]
        for j in range(_NFULL):
            pltpu.async_copy(
                table_hbm.at[idx, pl.ds(j * 128, 128)],
                rows.at[:, pl.ds(j * 128, 128)], sem)
        pltpu.async_copy(tail_hbm.at[idx], tail, sem)

    def wait_gathers(c, rows, tail, sem):
        idx = idx_v.at[pl.ds(c * _K, _K)]
        for j in range(_NFULL):
            pltpu.make_async_copy(
                table_hbm.at[idx, pl.ds(j * 128, 128)],
                rows.at[:, pl.ds(j * 128, 128)], sem).wait()
        pltpu.make_async_copy(tail_hbm.at[idx], tail, sem).wait()

    start_gathers(0, rows0, tail0, gsem0)
    start_gathers(1, rows1, tail1, gsem1)

    def step(c, rows, tail, sem):
        wait_gathers(c, rows, tail, sem)

        def merge_row(r, carry):
            for k in range(4):
                rows[r, pl.ds(_TAIL0 + 16 * k, 16)] = \
                    tail[r, pl.ds(16 * k, 16)]
            return carry
        lax.fori_loop(0, _K, merge_row, 0)

        pltpu.sync_copy(rows, out_hbm.at[pl.ds(base + c * _K, _K)])

        @pl.when(c + 2 < _NCH)
        def _():
            start_gathers(c + 2, rows, tail, sem)

    def pair(i, carry):
        step(2 * i, rows0, tail0, gsem0)
        step(2 * i + 1, rows1, tail1, gsem1)
        return carry

    lax.fori_loop(0, _NCH // 2, pair, 0)


@jax.jit
def _embed(ids_flat, font_table, tail_table):
    mesh = plsc.VectorSubcoreMesh(core_axis_name="c", subcore_axis_name="s")
    run = pl.kernel(
        _gather_body,
        out_type=jax.ShapeDtypeStruct((_B, EMBED_DIM), jnp.float32),
        mesh=mesh,
        scratch_types=[
            pltpu.VMEM((_BPW,), jnp.int32),
            pltpu.VMEM((_K, EMBED_DIM), jnp.float32),
            pltpu.VMEM((_K, EMBED_DIM), jnp.float32),
            pltpu.VMEM((_K, 128), jnp.float32),
            pltpu.VMEM((_K, 128), jnp.float32),
            pltpu.SemaphoreType.DMA,
            pltpu.SemaphoreType.DMA,
        ],
    )
    return run(font_table, tail_table, ids_flat)


def kernel(input_ids, font_table):
    ids_flat = input_ids.reshape(-1).astype(jnp.int32)
    tail_table = jnp.pad(font_table[:, _TAIL0:], ((0, 0), (0, 64)))
    out = _embed(ids_flat, font_table, tail_table)
    out3 = out.reshape(BATCH, SEQ, EMBED_DIM)
    return with_layout_constraint(out3, Layout(major_to_minor=(0, 1, 2)))
